# Initial kernel scaffold; baseline (speedup 1.0000x reference)
import functools

import jax
import jax.numpy as jnp
from jax import lax
from jax.experimental import pallas as pl
from jax.experimental.pallas import tpu as pltpu
from jax.experimental.pallas import tpu_sc as plsc

N = 10000
E = 320000
H = 128
EF = 16

NC = 2
NS = 16
NW = NC * NS
LANES = 16

EPW = E // NW
CSZ = 80
NCHUNK = EPW // CSZ
NPT = N // NS

_f32 = jnp.float32
_i32 = jnp.int32


def _silu(z):
    return z * jax.nn.sigmoid(z)


def _prep_body(h_ref, w_ref, a_ref, b_ref):
    ab = jnp.dot(h_ref[...], w_ref[...], preferred_element_type=_f32)
    a_ref[...] = ab[:, :H]
    b_ref[...] = ab[:, H:]


def _prep(h, w_cat):
    return pl.pallas_call(
        _prep_body,
        out_shape=[jax.ShapeDtypeStruct((N, H), _f32),
                   jax.ShapeDtypeStruct((N, H), _f32)],
    )(h, w_cat)


def _gather_body(a_hbm, b_hbm, row_hbm, col_hbm, ga_hbm, gb_hbm,
                 idxr, idxc, bufa, bufb, sema, semb):
    cid = lax.axis_index("c")
    sid = lax.axis_index("s")
    wid = cid * NS + sid

    def chunk(c, carry):
        pltpu.sync_copy(row_hbm.at[wid, c], idxr)
        pltpu.sync_copy(col_hbm.at[wid, c], idxc)
        cpa = pltpu.async_copy(a_hbm.at[idxr], bufa, sema)
        cpb = pltpu.async_copy(b_hbm.at[idxc], bufb, semb)
        cpa.wait()
        cpb.wait()
        pltpu.sync_copy(bufa, ga_hbm.at[wid, pl.ds(c * CSZ, CSZ)])
        pltpu.sync_copy(bufb, gb_hbm.at[wid, pl.ds(c * CSZ, CSZ)])
        return carry

    lax.fori_loop(0, NCHUNK, chunk, 0)


@functools.partial(
    pl.kernel,
    out_type=[jax.ShapeDtypeStruct((NW, EPW, H), _f32),
              jax.ShapeDtypeStruct((NW, EPW, H), _f32)],
    mesh=plsc.VectorSubcoreMesh(core_axis_name="c", subcore_axis_name="s",
                                num_cores=NC, num_subcores=NS),
    scratch_types=[
        pltpu.VMEM((CSZ,), _i32),
        pltpu.VMEM((CSZ,), _i32),
        pltpu.VMEM((CSZ, H), _f32),
        pltpu.VMEM((CSZ, H), _f32),
        pltpu.SemaphoreType.DMA,
        pltpu.SemaphoreType.DMA,
    ],
)
def _sc_gather(a_hbm, b_hbm, row_hbm, col_hbm, ga_hbm, gb_hbm,
               idxr, idxc, bufa, bufb, sema, semb):
    _gather_body(a_hbm, b_hbm, row_hbm, col_hbm, ga_hbm, gb_hbm,
                 idxr, idxc, bufa, bufb, sema, semb)


BE = 512


def _mlp_body(ga, gb, ea, wea, be1, we2, be2, wv1a, bv1a, wv1bt, bv1b,
              wv2a, bv2a, wv2bt, bv2b, out):
    z1 = ga[...] + gb[...] + be1[...] + jnp.dot(
        ea[...], wea[...], preferred_element_type=_f32)
    f1 = _silu(z1)
    z2 = jnp.dot(f1, we2[...], preferred_element_type=_f32) + be2[...]
    f2 = _silu(z2)
    t1 = _silu(jnp.dot(f2, wv1a[...], preferred_element_type=_f32) + bv1a[...])
    w1 = jnp.sum(t1 * wv1bt[...], axis=1, keepdims=True) + bv1b[...]
    t2 = _silu(jnp.dot(f2, wv2a[...], preferred_element_type=_f32) + bv2a[...])
    w2 = jnp.sum(t2 * wv2bt[...], axis=1, keepdims=True) + bv2b[...]
    out[...] = jnp.concatenate([w1, w2], axis=1)


def _mlp(ga, gb, ea, wea, be1, we2, be2, wv1a, bv1a, wv1bt, bv1b,
         wv2a, bv2a, wv2bt, bv2b):
    nblk = E // BE

    def full(shape):
        return pl.BlockSpec(shape, lambda i: (0,) * len(shape))

    return pl.pallas_call(
        _mlp_body,
        grid=(nblk,),
        in_specs=[
            pl.BlockSpec((BE, H), lambda i: (i, 0)),
            pl.BlockSpec((BE, H), lambda i: (i, 0)),
            pl.BlockSpec((BE, EF), lambda i: (i, 0)),
            full((EF, H)), full((1, H)), full((H, H)), full((1, H)),
            full((H, H)), full((1, H)), full((1, H)), full((1, 1)),
            full((H, H)), full((1, H)), full((1, H)), full((1, 1)),
        ],
        out_specs=pl.BlockSpec((BE, 2), lambda i: (i, 0)),
        out_shape=jax.ShapeDtypeStruct((E, 2), _f32),
    )(ga, gb, ea, wea, be1, we2, be2, wv1a, bv1a, wv1bt, bv1b,
      wv2a, bv2a, wv2bt, bv2b)


def _scatter_body(x_hbm, row_hbm, col_hbm, w_hbm, z_hbm, out_hbm,
                  xv, idxr, idxc, wbuf, vbuf, acc):
    cid = lax.axis_index("c")
    sid = lax.axis_index("s")
    wid = cid * NS + sid

    pltpu.sync_copy(z_hbm.at[pl.ds(sid * NPT, NPT)],
                    acc.at[pl.ds(sid * NPT, NPT)])
    pltpu.sync_copy(x_hbm, xv)
    plsc.subcore_barrier()

    lanes = lax.iota(_i32, (LANES,))
    c0 = jnp.zeros((LANES,), _i32)
    c1 = jnp.full((LANES,), 1, _i32)
    c2 = jnp.full((LANES,), 2, _i32)
    zf = jnp.zeros((LANES,), _f32)

    for v in range(CSZ // LANES):
        el = lanes + v * LANES
        plsc.store_scatter(vbuf, [el, jnp.full((LANES,), 3, _i32)], zf)
        plsc.store_scatter(vbuf, [el, jnp.full((LANES,), 7, _i32)], zf)

    def chunk(c, carry):
        pltpu.sync_copy(row_hbm.at[wid, c], idxr)
        pltpu.sync_copy(col_hbm.at[wid, c], idxc)
        pltpu.sync_copy(w_hbm.at[wid, pl.ds(c * CSZ, CSZ)], wbuf)
        for v in range(CSZ // LANES):
            el = lanes + v * LANES
            er = idxr[pl.ds(v * LANES, LANES)]
            ec = idxc[pl.ds(v * LANES, LANES)]
            w1 = plsc.load_gather(wbuf, [el, c0])
            w2 = plsc.load_gather(wbuf, [el, c1])
            d0 = plsc.load_gather(xv, [ec, c0]) - plsc.load_gather(xv, [er, c0])
            d1 = plsc.load_gather(xv, [ec, c1]) - plsc.load_gather(xv, [er, c1])
            d2 = plsc.load_gather(xv, [ec, c2]) - plsc.load_gather(xv, [er, c2])
            plsc.store_scatter(vbuf, [el, c0], d0 * w1)
            plsc.store_scatter(vbuf, [el, c1], d1 * w1)
            plsc.store_scatter(vbuf, [el, c2], d2 * w1)
            plsc.store_scatter(vbuf, [el, jnp.full((LANES,), 4, _i32)], d0 * w2)
            plsc.store_scatter(vbuf, [el, jnp.full((LANES,), 5, _i32)], d1 * w2)
            plsc.store_scatter(vbuf, [el, jnp.full((LANES,), 6, _i32)], d2 * w2)
        pltpu.sync_copy(vbuf, acc.at[idxr], add=True)
        return carry

    lax.fori_loop(0, NCHUNK, chunk, 0)
    plsc.subcore_barrier()

    @pl.when(sid == 0)
    def _():
        pltpu.sync_copy(acc, out_hbm.at[cid])


@functools.partial(
    pl.kernel,
    out_type=jax.ShapeDtypeStruct((NC, N, 8), _f32),
    mesh=plsc.VectorSubcoreMesh(core_axis_name="c", subcore_axis_name="s",
                                num_cores=NC, num_subcores=NS),
    scratch_types=[
        pltpu.VMEM((N, 3), _f32),
        pltpu.VMEM((CSZ,), _i32),
        pltpu.VMEM((CSZ,), _i32),
        pltpu.VMEM((CSZ, 2), _f32),
        pltpu.VMEM((CSZ, 8), _f32),
        pltpu.VMEM_SHARED((N, 8), _f32),
    ],
)
def _sc_scatter(x_hbm, row_hbm, col_hbm, w_hbm, z_hbm, out_hbm,
                xv, idxr, idxc, wbuf, vbuf, acc):
    _scatter_body(x_hbm, row_hbm, col_hbm, w_hbm, z_hbm, out_hbm,
                  xv, idxr, idxc, wbuf, vbuf, acc)


BN = 400


def _final_body(p_ref, out_ref):
    p = p_ref[...]
    v = p[0] + p[1]
    v1 = v[:, 0:3]
    v2 = v[:, 4:7]
    eps = jnp.float32(1e-12)
    n1 = jnp.sqrt(jnp.sum(v1 * v1, axis=1, keepdims=True))
    e1 = v1 / jnp.maximum(n1, eps)
    dot = jnp.sum(e1 * v2, axis=1, keepdims=True)
    pr = v2 - dot * e1
    n2 = jnp.sqrt(jnp.sum(pr * pr, axis=1, keepdims=True))
    e2 = pr / jnp.maximum(n2, eps)
    e1x, e1y, e1z = e1[:, 0:1], e1[:, 1:2], e1[:, 2:3]
    e2x, e2y, e2z = e2[:, 0:1], e2[:, 1:2], e2[:, 2:3]
    e3x = e1y * e2z - e1z * e2y
    e3y = e1z * e2x - e1x * e2z
    e3z = e1x * e2y - e1y * e2x
    out_ref[...] = jnp.concatenate(
        [e1x, e2x, e3x, e1y, e2y, e3y, e1z, e2z, e3z], axis=1)


def _final(partials):
    return pl.pallas_call(
        _final_body,
        grid=(N // BN,),
        in_specs=[pl.BlockSpec((NC, BN, 8), lambda i: (0, i, 0))],
        out_specs=pl.BlockSpec((BN, 9), lambda i: (i, 0)),
        out_shape=jax.ShapeDtypeStruct((N, 9), _f32),
    )(partials)


def kernel(h, x, edge_index, edge_attr, W_e1, b_e1, W_e2, b_e2,
           W_v1a, b_v1a, W_v1b, b_v1b, W_v2a, b_v2a, W_v2b, b_v2b):
    row = edge_index[0].astype(_i32).reshape(NW, NCHUNK, CSZ)
    col = edge_index[1].astype(_i32).reshape(NW, NCHUNK, CSZ)

    w_cat = jnp.concatenate([W_e1[:H], W_e1[H:2 * H]], axis=1)
    a_tab, b_tab = _prep(h, w_cat)

    ga, gb = _sc_gather(a_tab, b_tab, row, col)

    w_edges = _mlp(
        ga.reshape(E, H), gb.reshape(E, H), edge_attr,
        W_e1[2 * H:], b_e1.reshape(1, H),
        W_e2, b_e2.reshape(1, H),
        W_v1a, b_v1a.reshape(1, H), W_v1b.reshape(1, H), b_v1b.reshape(1, 1),
        W_v2a, b_v2a.reshape(1, H), W_v2b.reshape(1, H), b_v2b.reshape(1, 1),
    )

    zeros8 = jnp.zeros((N, 8), _f32)
    partials = _sc_scatter(x, row, col, w_edges.reshape(NW, EPW, 2), zeros8)

    out9 = _final(partials)
    return out9.reshape(N, 3, 3)


# speedup vs baseline: 2.7278x; 2.7278x over previous
import functools

import jax
import jax.numpy as jnp
from jax import lax
from jax.experimental import pallas as pl
from jax.experimental.pallas import tpu as pltpu
from jax.experimental.pallas import tpu_sc as plsc

N = 10000
E = 320000
H = 128
EF = 16

NC = 2
NS = 16
NW = NC * NS

EPW = E // NW
CSZ = 80
NCHUNK = EPW // CSZ
XP = 16

_f32 = jnp.float32
_i32 = jnp.int32


def _silu(z):
    return z * jax.nn.sigmoid(z)


def _prep_body(h_ref, w_ref, a_ref, b_ref):
    ab = jnp.dot(h_ref[...], w_ref[...], preferred_element_type=_f32)
    a_ref[...] = ab[:, :H]
    b_ref[...] = ab[:, H:]


def _prep(h, w_cat):
    return pl.pallas_call(
        _prep_body,
        out_shape=[jax.ShapeDtypeStruct((N, H), _f32),
                   jax.ShapeDtypeStruct((N, H), _f32)],
    )(h, w_cat)


def _gather_body(a_hbm, b_hbm, x_hbm, row_hbm, col_hbm,
                 ga_hbm, gb_hbm, xr_hbm, xc_hbm,
                 idxr, idxc, bufa, bufb, bufxr, bufxc, sema, semb, semx):
    cid = lax.axis_index("c")
    sid = lax.axis_index("s")
    wid = cid * NS + sid

    def chunk(c, carry):
        base = wid * EPW + c * CSZ
        pltpu.sync_copy(row_hbm.at[pl.ds(base, CSZ)], idxr)
        pltpu.sync_copy(col_hbm.at[pl.ds(base, CSZ)], idxc)
        cpa = pltpu.async_copy(a_hbm.at[idxr], bufa, sema)
        cpb = pltpu.async_copy(b_hbm.at[idxc], bufb, semb)
        cpxr = pltpu.async_copy(x_hbm.at[idxr], bufxr, semx)
        cpxc = pltpu.async_copy(x_hbm.at[idxc], bufxc, semx)
        cpa.wait()
        cpb.wait()
        cpxr.wait()
        cpxc.wait()
        pltpu.sync_copy(bufa, ga_hbm.at[wid, pl.ds(c * CSZ, CSZ)])
        pltpu.sync_copy(bufb, gb_hbm.at[wid, pl.ds(c * CSZ, CSZ)])
        pltpu.sync_copy(bufxr, xr_hbm.at[pl.ds(base, CSZ)])
        pltpu.sync_copy(bufxc, xc_hbm.at[pl.ds(base, CSZ)])
        return carry

    lax.fori_loop(0, NCHUNK, chunk, 0)


@functools.partial(
    pl.kernel,
    out_type=[jax.ShapeDtypeStruct((NW, EPW, H), _f32),
              jax.ShapeDtypeStruct((NW, EPW, H), _f32),
              jax.ShapeDtypeStruct((E, XP), _f32),
              jax.ShapeDtypeStruct((E, XP), _f32)],
    mesh=plsc.VectorSubcoreMesh(core_axis_name="c", subcore_axis_name="s",
                                num_cores=NC, num_subcores=NS),
    compiler_params=pltpu.CompilerParams(use_tc_tiling_on_sc=False),
    scratch_types=[
        pltpu.VMEM((CSZ,), _i32),
        pltpu.VMEM((CSZ,), _i32),
        pltpu.VMEM((CSZ, H), _f32),
        pltpu.VMEM((CSZ, H), _f32),
        pltpu.VMEM((CSZ, XP), _f32),
        pltpu.VMEM((CSZ, XP), _f32),
        pltpu.SemaphoreType.DMA,
        pltpu.SemaphoreType.DMA,
        pltpu.SemaphoreType.DMA,
    ],
)
def _sc_gather(a_hbm, b_hbm, x_hbm, row_hbm, col_hbm,
               ga_hbm, gb_hbm, xr_hbm, xc_hbm,
               idxr, idxc, bufa, bufb, bufxr, bufxc, sema, semb, semx):
    _gather_body(a_hbm, b_hbm, x_hbm, row_hbm, col_hbm,
                 ga_hbm, gb_hbm, xr_hbm, xc_hbm,
                 idxr, idxc, bufa, bufb, bufxr, bufxc, sema, semb, semx)


BE = 512


def _mlp_body(ga, gb, ea, xr, xc, wea, be1, we2, be2, wv1a, bv1a, wv1bt, bv1b,
              wv2a, bv2a, wv2bt, bv2b, out):
    z1 = ga[...] + gb[...] + be1[...] + jnp.dot(
        ea[...], wea[...], preferred_element_type=_f32)
    f1 = _silu(z1)
    z2 = jnp.dot(f1, we2[...], preferred_element_type=_f32) + be2[...]
    f2 = _silu(z2)
    t1 = _silu(jnp.dot(f2, wv1a[...], preferred_element_type=_f32) + bv1a[...])
    w1 = jnp.sum(t1 * wv1bt[...], axis=1, keepdims=True) + bv1b[...]
    t2 = _silu(jnp.dot(f2, wv2a[...], preferred_element_type=_f32) + bv2a[...])
    w2 = jnp.sum(t2 * wv2bt[...], axis=1, keepdims=True) + bv2b[...]
    d4 = (xc[...] - xr[...])[:, 0:4]
    out[...] = jnp.concatenate([d4 * w1, d4 * w2], axis=1)


def _mlp(ga, gb, ea, xr, xc, wea, be1, we2, be2, wv1a, bv1a, wv1bt, bv1b,
         wv2a, bv2a, wv2bt, bv2b):
    nblk = E // BE

    def full(shape):
        return pl.BlockSpec(shape, lambda i: (0,) * len(shape))

    return pl.pallas_call(
        _mlp_body,
        grid=(nblk,),
        in_specs=[
            pl.BlockSpec((BE, H), lambda i: (i, 0)),
            pl.BlockSpec((BE, H), lambda i: (i, 0)),
            pl.BlockSpec((BE, EF), lambda i: (i, 0)),
            pl.BlockSpec((BE, XP), lambda i: (i, 0)),
            pl.BlockSpec((BE, XP), lambda i: (i, 0)),
            full((EF, H)), full((1, H)), full((H, H)), full((1, H)),
            full((H, H)), full((1, H)), full((1, H)), full((1, 1)),
            full((H, H)), full((1, H)), full((1, H)), full((1, 1)),
        ],
        out_specs=pl.BlockSpec((BE, 8), lambda i: (i, 0)),
        out_shape=jax.ShapeDtypeStruct((E, 8), _f32),
    )(ga, gb, ea, xr, xc, wea, be1, we2, be2, wv1a, bv1a, wv1bt, bv1b,
      wv2a, bv2a, wv2bt, bv2b)


def _scatter_body(row_hbm, vec_hbm, z_hbm, out_hbm, idxr, vbuf, acc):
    cid = lax.axis_index("c")
    sid = lax.axis_index("s")
    wid = cid * NS + sid

    @pl.when(sid == 0)
    def _():
        pltpu.sync_copy(z_hbm, acc)

    plsc.subcore_barrier()

    def chunk(c, carry):
        base = wid * EPW + c * CSZ
        pltpu.sync_copy(row_hbm.at[pl.ds(base, CSZ)], idxr)
        pltpu.sync_copy(vec_hbm.at[pl.ds(base, CSZ)], vbuf)
        pltpu.sync_copy(vbuf, acc.at[idxr], add=True)
        return carry

    lax.fori_loop(0, NCHUNK, chunk, 0)
    plsc.subcore_barrier()

    @pl.when(sid == 0)
    def _():
        pltpu.sync_copy(acc, out_hbm.at[cid])


@functools.partial(
    pl.kernel,
    out_type=jax.ShapeDtypeStruct((NC, N, 8), _f32),
    mesh=plsc.VectorSubcoreMesh(core_axis_name="c", subcore_axis_name="s",
                                num_cores=NC, num_subcores=NS),
    compiler_params=pltpu.CompilerParams(use_tc_tiling_on_sc=False),
    scratch_types=[
        pltpu.VMEM((CSZ,), _i32),
        pltpu.VMEM((CSZ, 8), _f32),
        pltpu.VMEM_SHARED((N, 8), _f32),
    ],
)
def _sc_scatter(row_hbm, vec_hbm, z_hbm, out_hbm, idxr, vbuf, acc):
    _scatter_body(row_hbm, vec_hbm, z_hbm, out_hbm, idxr, vbuf, acc)


BN = 400


def _final_body(p_ref, out_ref):
    p = p_ref[...]
    v = p[0] + p[1]
    v1 = v[:, 0:3]
    v2 = v[:, 4:7]
    eps = jnp.float32(1e-12)
    n1 = jnp.sqrt(jnp.sum(v1 * v1, axis=1, keepdims=True))
    e1 = v1 / jnp.maximum(n1, eps)
    dot = jnp.sum(e1 * v2, axis=1, keepdims=True)
    pr = v2 - dot * e1
    n2 = jnp.sqrt(jnp.sum(pr * pr, axis=1, keepdims=True))
    e2 = pr / jnp.maximum(n2, eps)
    e1x, e1y, e1z = e1[:, 0:1], e1[:, 1:2], e1[:, 2:3]
    e2x, e2y, e2z = e2[:, 0:1], e2[:, 1:2], e2[:, 2:3]
    e3x = e1y * e2z - e1z * e2y
    e3y = e1z * e2x - e1x * e2z
    e3z = e1x * e2y - e1y * e2x
    out_ref[...] = jnp.concatenate(
        [e1x, e2x, e3x, e1y, e2y, e3y, e1z, e2z, e3z], axis=1)


def _final(partials):
    return pl.pallas_call(
        _final_body,
        grid=(N // BN,),
        in_specs=[pl.BlockSpec((NC, BN, 8), lambda i: (0, i, 0))],
        out_specs=pl.BlockSpec((BN, 9), lambda i: (i, 0)),
        out_shape=jax.ShapeDtypeStruct((N, 9), _f32),
    )(partials)


def kernel(h, x, edge_index, edge_attr, W_e1, b_e1, W_e2, b_e2,
           W_v1a, b_v1a, W_v1b, b_v1b, W_v2a, b_v2a, W_v2b, b_v2b):
    row = edge_index[0].astype(_i32)
    col = edge_index[1].astype(_i32)
    x16 = jnp.pad(x, ((0, 0), (0, XP - 3)))

    w_cat = jnp.concatenate([W_e1[:H], W_e1[H:2 * H]], axis=1)
    a_tab, b_tab = _prep(h, w_cat)

    ga, gb, xr, xc = _sc_gather(a_tab, b_tab, x16, row, col)

    vec = _mlp(
        ga.reshape(E, H), gb.reshape(E, H), edge_attr, xr, xc,
        W_e1[2 * H:], b_e1.reshape(1, H),
        W_e2, b_e2.reshape(1, H),
        W_v1a, b_v1a.reshape(1, H), W_v1b.reshape(1, H), b_v1b.reshape(1, 1),
        W_v2a, b_v2a.reshape(1, H), W_v2b.reshape(1, H), b_v2b.reshape(1, 1),
    )

    zeros8 = jnp.zeros((N, 8), _f32)
    partials = _sc_scatter(row, vec, zeros8)

    out9 = _final(partials)
    return out9.reshape(N, 3, 3)


# speedup vs baseline: 3.5741x; 1.3103x over previous
import functools

import jax
import jax.numpy as jnp
from jax import lax
from jax.experimental import pallas as pl
from jax.experimental.pallas import tpu as pltpu
from jax.experimental.pallas import tpu_sc as plsc

N = 10000
E = 320000
H = 128
EF = 16

NC = 2
NS = 16
NW = NC * NS

EPW = E // NW
CSZ = 80
NCHUNK = EPW // CSZ
XP = 16

_f32 = jnp.float32
_i32 = jnp.int32


def _silu(z):
    return z / (1.0 + jnp.exp(-z))


def _prep_body(h_ref, w_ref, a_ref, b_ref):
    ab = jnp.dot(h_ref[...], w_ref[...], preferred_element_type=_f32)
    a_ref[...] = ab[:, :H]
    b_ref[...] = ab[:, H:]


def _prep(h, w_cat):
    return pl.pallas_call(
        _prep_body,
        out_shape=[jax.ShapeDtypeStruct((N, H), _f32),
                   jax.ShapeDtypeStruct((N, H), _f32)],
    )(h, w_cat)


def _gather_body(a_hbm, b_hbm, x_hbm, row_hbm, col_hbm,
                 ga_hbm, gb_hbm, xr_hbm, xc_hbm,
                 rowv, colv, bufs, sems):
    cid = lax.axis_index("c")
    sid = lax.axis_index("s")
    wid = cid * NS + sid

    pltpu.sync_copy(row_hbm.at[pl.ds(wid * EPW, EPW)], rowv)
    pltpu.sync_copy(col_hbm.at[pl.ds(wid * EPW, EPW)], colv)

    def start(c, k):
        bufa, bufb, bufxr, bufxc = bufs[k]
        er = rowv.at[pl.ds(c * CSZ, CSZ)]
        ec = colv.at[pl.ds(c * CSZ, CSZ)]
        return [pltpu.async_copy(a_hbm.at[er], bufa, sems[k]),
                pltpu.async_copy(b_hbm.at[ec], bufb, sems[k]),
                pltpu.async_copy(x_hbm.at[er], bufxr, sems[k]),
                pltpu.async_copy(x_hbm.at[ec], bufxc, sems[k])]

    def drain(c, k):
        bufa, bufb, bufxr, bufxc = bufs[k]
        base = wid * EPW + c * CSZ
        pltpu.sync_copy(bufa, ga_hbm.at[wid, pl.ds(c * CSZ, CSZ)])
        pltpu.sync_copy(bufb, gb_hbm.at[wid, pl.ds(c * CSZ, CSZ)])
        pltpu.sync_copy(bufxr, xr_hbm.at[pl.ds(base, CSZ)])
        pltpu.sync_copy(bufxc, xc_hbm.at[pl.ds(base, CSZ)])

    def pair(k, carry):
        c0 = 2 * k
        cps0 = start(c0, 0)
        cps1 = start(c0 + 1, 1)
        for cp in cps0:
            cp.wait()
        drain(c0, 0)
        for cp in cps1:
            cp.wait()
        drain(c0 + 1, 1)
        return carry

    lax.fori_loop(0, NCHUNK // 2, pair, 0)
    cps = start(NCHUNK - 1, 0)
    for cp in cps:
        cp.wait()
    drain(NCHUNK - 1, 0)


@functools.partial(
    pl.kernel,
    out_type=[jax.ShapeDtypeStruct((NW, EPW, H), _f32),
              jax.ShapeDtypeStruct((NW, EPW, H), _f32),
              jax.ShapeDtypeStruct((E, XP), _f32),
              jax.ShapeDtypeStruct((E, XP), _f32)],
    mesh=plsc.VectorSubcoreMesh(core_axis_name="c", subcore_axis_name="s",
                                num_cores=NC, num_subcores=NS),
    compiler_params=pltpu.CompilerParams(use_tc_tiling_on_sc=False),
    scratch_types=[
        pltpu.VMEM((EPW,), _i32),
        pltpu.VMEM((EPW,), _i32),
        pltpu.VMEM((CSZ, H), _f32),
        pltpu.VMEM((CSZ, H), _f32),
        pltpu.VMEM((CSZ, XP), _f32),
        pltpu.VMEM((CSZ, XP), _f32),
        pltpu.VMEM((CSZ, H), _f32),
        pltpu.VMEM((CSZ, H), _f32),
        pltpu.VMEM((CSZ, XP), _f32),
        pltpu.VMEM((CSZ, XP), _f32),
        pltpu.SemaphoreType.DMA,
        pltpu.SemaphoreType.DMA,
    ],
)
def _sc_gather(a_hbm, b_hbm, x_hbm, row_hbm, col_hbm,
               ga_hbm, gb_hbm, xr_hbm, xc_hbm,
               rowv, colv, a0, b0, xr0, xc0, a1, b1, xr1, xc1, sem0, sem1):
    _gather_body(a_hbm, b_hbm, x_hbm, row_hbm, col_hbm,
                 ga_hbm, gb_hbm, xr_hbm, xc_hbm,
                 rowv, colv,
                 [(a0, b0, xr0, xc0), (a1, b1, xr1, xc1)], [sem0, sem1])


BE = 1280


def _mlp_body(ga, gb, ea, xr, xc, wea, be1, we2, be2, wvab, bvab, wvb, bvb,
              out):
    z1 = ga[...] + gb[...] + be1[...] + jnp.dot(
        ea[...], wea[...], preferred_element_type=_f32)
    f1 = _silu(z1)
    z2 = jnp.dot(f1, we2[...], preferred_element_type=_f32) + be2[...]
    f2 = _silu(z2)
    t12 = _silu(jnp.dot(f2, wvab[...], preferred_element_type=_f32)
                + bvab[...])
    w12 = jnp.dot(t12, wvb[...], preferred_element_type=_f32) + bvb[...]
    w1 = w12[:, 0:1]
    w2 = w12[:, 1:2]
    d4 = (xc[...] - xr[...])[:, 0:4]
    out[...] = jnp.concatenate([d4 * w1, d4 * w2], axis=1)


def _mlp(ga, gb, ea, xr, xc, wea, be1, we2, be2, wvab, bvab, wvb, bvb):
    nblk = E // BE

    def full(shape):
        return pl.BlockSpec(shape, lambda i: (0,) * len(shape))

    return pl.pallas_call(
        _mlp_body,
        grid=(nblk,),
        in_specs=[
            pl.BlockSpec((BE, H), lambda i: (i, 0)),
            pl.BlockSpec((BE, H), lambda i: (i, 0)),
            pl.BlockSpec((BE, EF), lambda i: (i, 0)),
            pl.BlockSpec((BE, XP), lambda i: (i, 0)),
            pl.BlockSpec((BE, XP), lambda i: (i, 0)),
            full((EF, H)), full((1, H)), full((H, H)), full((1, H)),
            full((H, 2 * H)), full((1, 2 * H)), full((2 * H, 2)),
            full((1, 2)),
        ],
        out_specs=pl.BlockSpec((BE, 8), lambda i: (i, 0)),
        out_shape=jax.ShapeDtypeStruct((E, 8), _f32),
    )(ga, gb, ea, xr, xc, wea, be1, we2, be2, wvab, bvab, wvb, bvb)


def _scatter_body(row_hbm, vec_hbm, z_hbm, out_hbm, idxr, vbuf, acc):
    cid = lax.axis_index("c")
    sid = lax.axis_index("s")
    wid = cid * NS + sid

    @pl.when(sid == 0)
    def _():
        pltpu.sync_copy(z_hbm, acc)

    plsc.subcore_barrier()

    def chunk(c, carry):
        base = wid * EPW + c * CSZ
        pltpu.sync_copy(row_hbm.at[pl.ds(base, CSZ)], idxr)
        pltpu.sync_copy(vec_hbm.at[pl.ds(base, CSZ)], vbuf)
        pltpu.sync_copy(vbuf, acc.at[idxr], add=True)
        return carry

    lax.fori_loop(0, NCHUNK, chunk, 0)
    plsc.subcore_barrier()

    @pl.when(sid == 0)
    def _():
        pltpu.sync_copy(acc, out_hbm.at[cid])


@functools.partial(
    pl.kernel,
    out_type=jax.ShapeDtypeStruct((NC, N, 8), _f32),
    mesh=plsc.VectorSubcoreMesh(core_axis_name="c", subcore_axis_name="s",
                                num_cores=NC, num_subcores=NS),
    compiler_params=pltpu.CompilerParams(use_tc_tiling_on_sc=False),
    scratch_types=[
        pltpu.VMEM((CSZ,), _i32),
        pltpu.VMEM((CSZ, 8), _f32),
        pltpu.VMEM_SHARED((N, 8), _f32),
    ],
)
def _sc_scatter(row_hbm, vec_hbm, z_hbm, out_hbm, idxr, vbuf, acc):
    _scatter_body(row_hbm, vec_hbm, z_hbm, out_hbm, idxr, vbuf, acc)


BN = 400


def _final_body(p_ref, out_ref):
    p = p_ref[...]
    v = p[0] + p[1]
    v1 = v[:, 0:3]
    v2 = v[:, 4:7]
    eps = jnp.float32(1e-12)
    n1 = jnp.sqrt(jnp.sum(v1 * v1, axis=1, keepdims=True))
    e1 = v1 / jnp.maximum(n1, eps)
    dot = jnp.sum(e1 * v2, axis=1, keepdims=True)
    pr = v2 - dot * e1
    n2 = jnp.sqrt(jnp.sum(pr * pr, axis=1, keepdims=True))
    e2 = pr / jnp.maximum(n2, eps)
    e1x, e1y, e1z = e1[:, 0:1], e1[:, 1:2], e1[:, 2:3]
    e2x, e2y, e2z = e2[:, 0:1], e2[:, 1:2], e2[:, 2:3]
    e3x = e1y * e2z - e1z * e2y
    e3y = e1z * e2x - e1x * e2z
    e3z = e1x * e2y - e1y * e2x
    out_ref[...] = jnp.concatenate(
        [e1x, e2x, e3x, e1y, e2y, e3y, e1z, e2z, e3z], axis=1)


def _final(partials):
    return pl.pallas_call(
        _final_body,
        grid=(N // BN,),
        in_specs=[pl.BlockSpec((NC, BN, 8), lambda i: (0, i, 0))],
        out_specs=pl.BlockSpec((BN, 9), lambda i: (i, 0)),
        out_shape=jax.ShapeDtypeStruct((N, 9), _f32),
    )(partials)


def kernel(h, x, edge_index, edge_attr, W_e1, b_e1, W_e2, b_e2,
           W_v1a, b_v1a, W_v1b, b_v1b, W_v2a, b_v2a, W_v2b, b_v2b):
    row = edge_index[0].astype(_i32)
    col = edge_index[1].astype(_i32)
    x16 = jnp.pad(x, ((0, 0), (0, XP - 3)))

    w_cat = jnp.concatenate([W_e1[:H], W_e1[H:2 * H]], axis=1)
    a_tab, b_tab = _prep(h, w_cat)

    ga, gb, xr, xc = _sc_gather(a_tab, b_tab, x16, row, col)

    wvab = jnp.concatenate([W_v1a, W_v2a], axis=1)
    bvab = jnp.concatenate([b_v1a, b_v2a]).reshape(1, 2 * H)
    z1c = jnp.zeros((H, 1), _f32)
    wvb = jnp.concatenate(
        [jnp.concatenate([W_v1b, z1c], axis=1),
         jnp.concatenate([z1c, W_v2b], axis=1)], axis=0)
    bvb = jnp.stack([b_v1b[0], b_v2b[0]]).reshape(1, 2)

    vec = _mlp(
        ga.reshape(E, H), gb.reshape(E, H), edge_attr, xr, xc,
        W_e1[2 * H:], b_e1.reshape(1, H),
        W_e2, b_e2.reshape(1, H),
        wvab, bvab, wvb, bvb,
    )

    zeros8 = jnp.zeros((N, 8), _f32)
    partials = _sc_scatter(row, vec, zeros8)

    out9 = _final(partials)
    return out9.reshape(N, 3, 3)


# speedup vs baseline: 3.8092x; 1.0658x over previous
import functools

import jax
import jax.numpy as jnp
from jax import lax
from jax.experimental import pallas as pl
from jax.experimental.pallas import tpu as pltpu
from jax.experimental.pallas import tpu_sc as plsc

N = 10000
E = 320000
H = 128
EF = 16

NC = 2
NS = 16
NW = NC * NS

EPW = E // NW
CSZ = 80
NCHUNK = EPW // CSZ
XP = 16

_f32 = jnp.float32
_i32 = jnp.int32


def _silu(z):
    return z / (1.0 + jnp.exp(-z))


def _prep_body(h_ref, w_ref, a_ref, b_ref):
    ab = jnp.dot(h_ref[...], w_ref[...], preferred_element_type=_f32)
    a_ref[...] = ab[:, :H]
    b_ref[...] = ab[:, H:]


def _prep(h, w_cat):
    return pl.pallas_call(
        _prep_body,
        out_shape=[jax.ShapeDtypeStruct((N, H), _f32),
                   jax.ShapeDtypeStruct((N, H), _f32)],
    )(h, w_cat)


def _gather_body(a_hbm, b_hbm, x_hbm, row_hbm, col_hbm,
                 ga_hbm, gb_hbm, xr_hbm, xc_hbm,
                 rowv, colv, bufs, sems):
    cid = lax.axis_index("c")
    sid = lax.axis_index("s")
    wid = cid * NS + sid

    pltpu.sync_copy(row_hbm.at[pl.ds(wid * EPW, EPW)], rowv)
    pltpu.sync_copy(col_hbm.at[pl.ds(wid * EPW, EPW)], colv)

    def start(c, k):
        bufa, bufb, bufxr, bufxc = bufs[k]
        er = rowv.at[pl.ds(c * CSZ, CSZ)]
        ec = colv.at[pl.ds(c * CSZ, CSZ)]
        return [pltpu.async_copy(a_hbm.at[er], bufa, sems[k]),
                pltpu.async_copy(b_hbm.at[ec], bufb, sems[k]),
                pltpu.async_copy(x_hbm.at[er], bufxr, sems[k]),
                pltpu.async_copy(x_hbm.at[ec], bufxc, sems[k])]

    def drain(c, k):
        bufa, bufb, bufxr, bufxc = bufs[k]
        base = wid * EPW + c * CSZ
        pltpu.sync_copy(bufa, ga_hbm.at[pl.ds(base, CSZ)])
        pltpu.sync_copy(bufb, gb_hbm.at[pl.ds(base, CSZ)])
        pltpu.sync_copy(bufxr, xr_hbm.at[pl.ds(base, CSZ)])
        pltpu.sync_copy(bufxc, xc_hbm.at[pl.ds(base, CSZ)])

    def pair(k, carry):
        c0 = 2 * k
        cps0 = start(c0, 0)
        cps1 = start(c0 + 1, 1)
        for cp in cps0:
            cp.wait()
        drain(c0, 0)
        for cp in cps1:
            cp.wait()
        drain(c0 + 1, 1)
        return carry

    lax.fori_loop(0, NCHUNK // 2, pair, 0)
    cps = start(NCHUNK - 1, 0)
    for cp in cps:
        cp.wait()
    drain(NCHUNK - 1, 0)


@functools.partial(
    pl.kernel,
    out_type=[jax.ShapeDtypeStruct((E, H), _f32),
              jax.ShapeDtypeStruct((E, H), _f32),
              jax.ShapeDtypeStruct((E, XP), _f32),
              jax.ShapeDtypeStruct((E, XP), _f32)],
    mesh=plsc.VectorSubcoreMesh(core_axis_name="c", subcore_axis_name="s",
                                num_cores=NC, num_subcores=NS),
    compiler_params=pltpu.CompilerParams(use_tc_tiling_on_sc=False),
    scratch_types=[
        pltpu.VMEM((EPW,), _i32),
        pltpu.VMEM((EPW,), _i32),
        pltpu.VMEM((CSZ, H), _f32),
        pltpu.VMEM((CSZ, H), _f32),
        pltpu.VMEM((CSZ, XP), _f32),
        pltpu.VMEM((CSZ, XP), _f32),
        pltpu.VMEM((CSZ, H), _f32),
        pltpu.VMEM((CSZ, H), _f32),
        pltpu.VMEM((CSZ, XP), _f32),
        pltpu.VMEM((CSZ, XP), _f32),
        pltpu.SemaphoreType.DMA,
        pltpu.SemaphoreType.DMA,
    ],
)
def _sc_gather(a_hbm, b_hbm, x_hbm, row_hbm, col_hbm,
               ga_hbm, gb_hbm, xr_hbm, xc_hbm,
               rowv, colv, a0, b0, xr0, xc0, a1, b1, xr1, xc1, sem0, sem1):
    _gather_body(a_hbm, b_hbm, x_hbm, row_hbm, col_hbm,
                 ga_hbm, gb_hbm, xr_hbm, xc_hbm,
                 rowv, colv,
                 [(a0, b0, xr0, xc0), (a1, b1, xr1, xc1)], [sem0, sem1])


BE = 1280


def _mlp_body(ga, gb, ea, xr, xc, wea, be1, we2, be2, wvab, bvab, wvb, bvb,
              out):
    z1 = ga[...] + gb[...] + be1[...] + jnp.dot(
        ea[...], wea[...], preferred_element_type=_f32)
    f1 = _silu(z1)
    z2 = jnp.dot(f1, we2[...], preferred_element_type=_f32) + be2[...]
    f2 = _silu(z2)
    t12 = _silu(jnp.dot(f2, wvab[...], preferred_element_type=_f32)
                + bvab[...])
    w12 = jnp.dot(t12, wvb[...], preferred_element_type=_f32) + bvb[...]
    w1 = w12[:, 0:1]
    w2 = w12[:, 1:2]
    d4 = (xc[...] - xr[...])[:, 0:4]
    out[...] = jnp.concatenate([d4 * w1, d4 * w2], axis=1)


def _mlp(ga, gb, ea, xr, xc, wea, be1, we2, be2, wvab, bvab, wvb, bvb):
    nblk = E // BE

    def full(shape):
        return pl.BlockSpec(shape, lambda i: (0,) * len(shape))

    return pl.pallas_call(
        _mlp_body,
        grid=(nblk,),
        in_specs=[
            pl.BlockSpec((BE, H), lambda i: (i, 0)),
            pl.BlockSpec((BE, H), lambda i: (i, 0)),
            pl.BlockSpec((BE, EF), lambda i: (i, 0)),
            pl.BlockSpec((BE, XP), lambda i: (i, 0)),
            pl.BlockSpec((BE, XP), lambda i: (i, 0)),
            full((EF, H)), full((1, H)), full((H, H)), full((1, H)),
            full((H, 2 * H)), full((1, 2 * H)), full((2 * H, 2)),
            full((1, 2)),
        ],
        out_specs=pl.BlockSpec((BE, 8), lambda i: (i, 0)),
        out_shape=jax.ShapeDtypeStruct((E, 8), _f32),
    )(ga, gb, ea, xr, xc, wea, be1, we2, be2, wvab, bvab, wvb, bvb)


def _scatter_body(row3_hbm, vec_hbm, z_hbm, out_hbm, idxv, vb0, vb1,
                  sem0, sem1, acc):
    cid = lax.axis_index("c")
    sid = lax.axis_index("s")
    wid = cid * NS + sid

    @pl.when(sid == 0)
    def _():
        pltpu.sync_copy(z_hbm, acc)

    pltpu.sync_copy(row3_hbm.at[wid], idxv)
    plsc.subcore_barrier()

    def vload(c, buf, sem):
        base = wid * EPW + c * CSZ
        return pltpu.async_copy(vec_hbm.at[pl.ds(base, CSZ)], buf, sem)

    def pair(k, carry):
        c0 = 2 * k
        cp0 = vload(c0, vb0, sem0)
        cp1 = vload(c0 + 1, vb1, sem1)
        cp0.wait()
        pltpu.sync_copy(vb0, acc.at[idxv.at[c0]], add=True)
        cp1.wait()
        pltpu.sync_copy(vb1, acc.at[idxv.at[c0 + 1]], add=True)
        return carry

    lax.fori_loop(0, NCHUNK // 2, pair, 0)
    cp = vload(NCHUNK - 1, vb0, sem0)
    cp.wait()
    pltpu.sync_copy(vb0, acc.at[idxv.at[NCHUNK - 1]], add=True)

    plsc.subcore_barrier()

    @pl.when(sid == 0)
    def _():
        pltpu.sync_copy(acc, out_hbm.at[cid])


@functools.partial(
    pl.kernel,
    out_type=jax.ShapeDtypeStruct((NC, N, 8), _f32),
    mesh=plsc.VectorSubcoreMesh(core_axis_name="c", subcore_axis_name="s",
                                num_cores=NC, num_subcores=NS),
    compiler_params=pltpu.CompilerParams(use_tc_tiling_on_sc=False),
    scratch_types=[
        pltpu.VMEM((NCHUNK, CSZ), _i32),
        pltpu.VMEM((CSZ, 8), _f32),
        pltpu.VMEM((CSZ, 8), _f32),
        pltpu.SemaphoreType.DMA,
        pltpu.SemaphoreType.DMA,
        pltpu.VMEM_SHARED((N, 8), _f32),
    ],
)
def _sc_scatter(row3_hbm, vec_hbm, z_hbm, out_hbm, idxv, vb0, vb1,
                sem0, sem1, acc):
    _scatter_body(row3_hbm, vec_hbm, z_hbm, out_hbm, idxv, vb0, vb1,
                  sem0, sem1, acc)


BN = 400


def _final_body(p_ref, out_ref):
    p = p_ref[...]
    v = p[0] + p[1]
    v1 = v[:, 0:3]
    v2 = v[:, 4:7]
    eps = jnp.float32(1e-12)
    n1 = jnp.sqrt(jnp.sum(v1 * v1, axis=1, keepdims=True))
    e1 = v1 / jnp.maximum(n1, eps)
    dot = jnp.sum(e1 * v2, axis=1, keepdims=True)
    pr = v2 - dot * e1
    n2 = jnp.sqrt(jnp.sum(pr * pr, axis=1, keepdims=True))
    e2 = pr / jnp.maximum(n2, eps)
    e1x, e1y, e1z = e1[:, 0:1], e1[:, 1:2], e1[:, 2:3]
    e2x, e2y, e2z = e2[:, 0:1], e2[:, 1:2], e2[:, 2:3]
    e3x = e1y * e2z - e1z * e2y
    e3y = e1z * e2x - e1x * e2z
    e3z = e1x * e2y - e1y * e2x
    out_ref[...] = jnp.concatenate(
        [e1x, e2x, e3x, e1y, e2y, e3y, e1z, e2z, e3z], axis=1)


def _final(partials):
    return pl.pallas_call(
        _final_body,
        grid=(N // BN,),
        in_specs=[pl.BlockSpec((NC, BN, 8), lambda i: (0, i, 0))],
        out_specs=pl.BlockSpec((BN, 9), lambda i: (i, 0)),
        out_shape=jax.ShapeDtypeStruct((N, 9), _f32),
    )(partials)


def kernel(h, x, edge_index, edge_attr, W_e1, b_e1, W_e2, b_e2,
           W_v1a, b_v1a, W_v1b, b_v1b, W_v2a, b_v2a, W_v2b, b_v2b):
    row = edge_index[0].astype(_i32)
    col = edge_index[1].astype(_i32)
    x16 = jnp.pad(x, ((0, 0), (0, XP - 3)))

    w_cat = jnp.concatenate([W_e1[:H], W_e1[H:2 * H]], axis=1)
    a_tab, b_tab = _prep(h, w_cat)

    ga, gb, xr, xc = _sc_gather(a_tab, b_tab, x16, row, col)

    wvab = jnp.concatenate([W_v1a, W_v2a], axis=1)
    bvab = jnp.concatenate([b_v1a, b_v2a]).reshape(1, 2 * H)
    z1c = jnp.zeros((H, 1), _f32)
    wvb = jnp.concatenate(
        [jnp.concatenate([W_v1b, z1c], axis=1),
         jnp.concatenate([z1c, W_v2b], axis=1)], axis=0)
    bvb = jnp.stack([b_v1b[0], b_v2b[0]]).reshape(1, 2)

    vec = _mlp(
        ga, gb, edge_attr, xr, xc,
        W_e1[2 * H:], b_e1.reshape(1, H),
        W_e2, b_e2.reshape(1, H),
        wvab, bvab, wvb, bvb,
    )

    zeros8 = jnp.zeros((N, 8), _f32)
    partials = _sc_scatter(row.reshape(NW, NCHUNK, CSZ), vec, zeros8)

    out9 = _final(partials)
    return out9.reshape(N, 3, 3)


# speedup vs baseline: 4.1965x; 1.1017x over previous
import functools

import jax
import jax.numpy as jnp
from jax import lax
from jax.experimental import pallas as pl
from jax.experimental.pallas import tpu as pltpu
from jax.experimental.pallas import tpu_sc as plsc

N = 10000
E = 320000
H = 128
EF = 16

NC = 2
NS = 16
NW = NC * NS

S = 5
ES = E // S
EPW = ES // NW
CSZ = 80
NCHUNK = EPW // CSZ
XP = 16

_f32 = jnp.float32
_i32 = jnp.int32


def _silu(z):
    return z / (1.0 + jnp.exp(-z))


def _prep_body(h_ref, w_ref, a_ref, b_ref):
    ab = jnp.dot(h_ref[...], w_ref[...], preferred_element_type=_f32)
    a_ref[...] = ab[:, :H]
    b_ref[...] = ab[:, H:]


def _prep(h, w_cat):
    return pl.pallas_call(
        _prep_body,
        out_shape=[jax.ShapeDtypeStruct((N, H), _f32),
                   jax.ShapeDtypeStruct((N, H), _f32)],
    )(h, w_cat)


def _gather_body(a_hbm, b_hbm, x_hbm, row_hbm, col_hbm,
                 ga_hbm, gb_hbm, xc_hbm, rowv, colv, bufs, sems):
    cid = lax.axis_index("c")
    sid = lax.axis_index("s")
    wid = cid * NS + sid

    pltpu.sync_copy(row_hbm.at[pl.ds(wid * EPW, EPW)], rowv)
    pltpu.sync_copy(col_hbm.at[pl.ds(wid * EPW, EPW)], colv)

    def start(c, k):
        bufa, bufb, bufxc = bufs[k]
        er = rowv.at[pl.ds(c * CSZ, CSZ)]
        ec = colv.at[pl.ds(c * CSZ, CSZ)]
        return [pltpu.async_copy(a_hbm.at[er], bufa, sems[k]),
                pltpu.async_copy(b_hbm.at[ec], bufb, sems[k]),
                pltpu.async_copy(x_hbm.at[ec], bufxc, sems[k])]

    def drain(c, k):
        bufa, bufb, bufxc = bufs[k]
        base = wid * EPW + c * CSZ
        pltpu.sync_copy(bufa, ga_hbm.at[pl.ds(base, CSZ)])
        pltpu.sync_copy(bufb, gb_hbm.at[pl.ds(base, CSZ)])
        pltpu.sync_copy(bufxc, xc_hbm.at[pl.ds(base, CSZ)])

    def pair(k, carry):
        c0 = 2 * k
        cps0 = start(c0, 0)
        cps1 = start(c0 + 1, 1)
        for cp in cps0:
            cp.wait()
        drain(c0, 0)
        for cp in cps1:
            cp.wait()
        drain(c0 + 1, 1)
        return carry

    lax.fori_loop(0, NCHUNK // 2, pair, 0)
    cps = start(NCHUNK - 1, 0)
    for cp in cps:
        cp.wait()
    drain(NCHUNK - 1, 0)


@functools.partial(
    pl.kernel,
    out_type=[jax.ShapeDtypeStruct((ES, H), _f32),
              jax.ShapeDtypeStruct((ES, H), _f32),
              jax.ShapeDtypeStruct((ES, XP), _f32)],
    mesh=plsc.VectorSubcoreMesh(core_axis_name="c", subcore_axis_name="s",
                                num_cores=NC, num_subcores=NS),
    compiler_params=pltpu.CompilerParams(use_tc_tiling_on_sc=False),
    scratch_types=[
        pltpu.VMEM((EPW,), _i32),
        pltpu.VMEM((EPW,), _i32),
        pltpu.VMEM((CSZ, H), _f32),
        pltpu.VMEM((CSZ, H), _f32),
        pltpu.VMEM((CSZ, XP), _f32),
        pltpu.VMEM((CSZ, H), _f32),
        pltpu.VMEM((CSZ, H), _f32),
        pltpu.VMEM((CSZ, XP), _f32),
        pltpu.SemaphoreType.DMA,
        pltpu.SemaphoreType.DMA,
    ],
)
def _sc_gather(a_hbm, b_hbm, x_hbm, row_hbm, col_hbm,
               ga_hbm, gb_hbm, xc_hbm,
               rowv, colv, a0, b0, xc0, a1, b1, xc1, sem0, sem1):
    _gather_body(a_hbm, b_hbm, x_hbm, row_hbm, col_hbm,
                 ga_hbm, gb_hbm, xc_hbm, rowv, colv,
                 [(a0, b0, xc0), (a1, b1, xc1)], [sem0, sem1])


BE = 1280


def _mlp_body(ga, gb, ea, xc, wea, be1, we2, be2, wvab, bvab, wvb, bvb, out):
    z1 = ga[...] + gb[...] + be1[...] + jnp.dot(
        ea[...], wea[...], preferred_element_type=_f32)
    f1 = _silu(z1)
    z2 = jnp.dot(f1, we2[...], preferred_element_type=_f32) + be2[...]
    f2 = _silu(z2)
    t12 = _silu(jnp.dot(f2, wvab[...], preferred_element_type=_f32)
                + bvab[...])
    w12 = jnp.dot(t12, wvb[...], preferred_element_type=_f32) + bvb[...]
    w1 = w12[:, 0:1]
    w2 = w12[:, 1:2]
    d4 = jnp.concatenate(
        [xc[...][:, 0:3], jnp.ones((BE, 1), _f32)], axis=1)
    out[...] = jnp.concatenate([d4 * w1, d4 * w2], axis=1)


def _mlp(ga, gb, ea, xc, wea, be1, we2, be2, wvab, bvab, wvb, bvb):
    nblk = ES // BE

    def full(shape):
        return pl.BlockSpec(shape, lambda i: (0,) * len(shape))

    return pl.pallas_call(
        _mlp_body,
        grid=(nblk,),
        in_specs=[
            pl.BlockSpec((BE, H), lambda i: (i, 0)),
            pl.BlockSpec((BE, H), lambda i: (i, 0)),
            pl.BlockSpec((BE, EF), lambda i: (i, 0)),
            pl.BlockSpec((BE, XP), lambda i: (i, 0)),
            full((EF, H)), full((1, H)), full((H, H)), full((1, H)),
            full((H, 2 * H)), full((1, 2 * H)), full((2 * H, 2)),
            full((1, 2)),
        ],
        out_specs=pl.BlockSpec((BE, 8), lambda i: (i, 0)),
        out_shape=jax.ShapeDtypeStruct((ES, 8), _f32),
    )(ga, gb, ea, xc, wea, be1, we2, be2, wvab, bvab, wvb, bvb)


def _scatter_body(row4_hbm, vecs_hbm, z_hbm, out_hbm, idxv, vb0, vb1,
                  sem0, sem1, acc):
    cid = lax.axis_index("c")
    sid = lax.axis_index("s")
    wid = cid * NS + sid

    @pl.when(sid == 0)
    def _():
        pltpu.sync_copy(z_hbm, acc)

    plsc.subcore_barrier()

    for s in range(S):
        vec_hbm = vecs_hbm[s]
        pltpu.sync_copy(row4_hbm.at[s, wid], idxv)

        def vload(c, buf, sem):
            base = wid * EPW + c * CSZ
            return pltpu.async_copy(vec_hbm.at[pl.ds(base, CSZ)], buf, sem)

        def pair(k, carry):
            c0 = 2 * k
            cp0 = vload(c0, vb0, sem0)
            cp1 = vload(c0 + 1, vb1, sem1)
            cp0.wait()
            pltpu.sync_copy(vb0, acc.at[idxv.at[c0]], add=True)
            cp1.wait()
            pltpu.sync_copy(vb1, acc.at[idxv.at[c0 + 1]], add=True)
            return carry

        lax.fori_loop(0, NCHUNK // 2, pair, 0)
        cp = vload(NCHUNK - 1, vb0, sem0)
        cp.wait()
        pltpu.sync_copy(vb0, acc.at[idxv.at[NCHUNK - 1]], add=True)

    plsc.subcore_barrier()

    @pl.when(sid == 0)
    def _():
        pltpu.sync_copy(acc, out_hbm.at[cid])


@functools.partial(
    pl.kernel,
    out_type=jax.ShapeDtypeStruct((NC, N, 8), _f32),
    mesh=plsc.VectorSubcoreMesh(core_axis_name="c", subcore_axis_name="s",
                                num_cores=NC, num_subcores=NS),
    compiler_params=pltpu.CompilerParams(use_tc_tiling_on_sc=False),
    scratch_types=[
        pltpu.VMEM((NCHUNK, CSZ), _i32),
        pltpu.VMEM((CSZ, 8), _f32),
        pltpu.VMEM((CSZ, 8), _f32),
        pltpu.SemaphoreType.DMA,
        pltpu.SemaphoreType.DMA,
        pltpu.VMEM_SHARED((N, 8), _f32),
    ],
)
def _sc_scatter(row4_hbm, v0, v1, v2, v3, v4, z_hbm, out_hbm,
                idxv, vb0, vb1, sem0, sem1, acc):
    _scatter_body(row4_hbm, [v0, v1, v2, v3, v4], z_hbm, out_hbm,
                  idxv, vb0, vb1, sem0, sem1, acc)


BN = 400


def _final_body(p_ref, x_ref, out_ref):
    p = p_ref[...]
    xb = x_ref[...]
    v = p[0] + p[1]
    v1 = v[:, 0:3] - xb * v[:, 3:4]
    v2 = v[:, 4:7] - xb * v[:, 7:8]
    eps = jnp.float32(1e-12)
    n1 = jnp.sqrt(jnp.sum(v1 * v1, axis=1, keepdims=True))
    e1 = v1 / jnp.maximum(n1, eps)
    dot = jnp.sum(e1 * v2, axis=1, keepdims=True)
    pr = v2 - dot * e1
    n2 = jnp.sqrt(jnp.sum(pr * pr, axis=1, keepdims=True))
    e2 = pr / jnp.maximum(n2, eps)
    e1x, e1y, e1z = e1[:, 0:1], e1[:, 1:2], e1[:, 2:3]
    e2x, e2y, e2z = e2[:, 0:1], e2[:, 1:2], e2[:, 2:3]
    e3x = e1y * e2z - e1z * e2y
    e3y = e1z * e2x - e1x * e2z
    e3z = e1x * e2y - e1y * e2x
    out_ref[...] = jnp.concatenate(
        [e1x, e2x, e3x, e1y, e2y, e3y, e1z, e2z, e3z], axis=1)


def _final(partials, x):
    return pl.pallas_call(
        _final_body,
        grid=(N // BN,),
        in_specs=[pl.BlockSpec((NC, BN, 8), lambda i: (0, i, 0)),
                  pl.BlockSpec((BN, 3), lambda i: (i, 0))],
        out_specs=pl.BlockSpec((BN, 9), lambda i: (i, 0)),
        out_shape=jax.ShapeDtypeStruct((N, 9), _f32),
    )(partials, x)


def kernel(h, x, edge_index, edge_attr, W_e1, b_e1, W_e2, b_e2,
           W_v1a, b_v1a, W_v1b, b_v1b, W_v2a, b_v2a, W_v2b, b_v2b):
    row = edge_index[0].astype(_i32)
    col = edge_index[1].astype(_i32)
    row2 = row.reshape(S, ES)
    col2 = col.reshape(S, ES)
    ea2 = edge_attr.reshape(S, ES, EF)
    x16 = jnp.pad(x, ((0, 0), (0, XP - 3)))

    w_cat = jnp.concatenate([W_e1[:H], W_e1[H:2 * H]], axis=1)
    a_tab, b_tab = _prep(h, w_cat)

    wea = W_e1[2 * H:]
    be1 = b_e1.reshape(1, H)
    be2 = b_e2.reshape(1, H)
    wvab = jnp.concatenate([W_v1a, W_v2a], axis=1)
    bvab = jnp.concatenate([b_v1a, b_v2a]).reshape(1, 2 * H)
    z1c = jnp.zeros((H, 1), _f32)
    wvb = jnp.concatenate(
        [jnp.concatenate([W_v1b, z1c], axis=1),
         jnp.concatenate([z1c, W_v2b], axis=1)], axis=0)
    bvb = jnp.stack([b_v1b[0], b_v2b[0]]).reshape(1, 2)

    vecs = []
    for s in range(S):
        ga, gb, xc = _sc_gather(a_tab, b_tab, x16, row2[s], col2[s])
        vecs.append(_mlp(ga, gb, ea2[s], xc,
                         wea, be1, W_e2, be2, wvab, bvab, wvb, bvb))

    zeros8 = jnp.zeros((N, 8), _f32)
    partials = _sc_scatter(row.reshape(S, NW, NCHUNK, CSZ), *vecs, zeros8)

    out9 = _final(partials, x)
    return out9.reshape(N, 3, 3)


# speedup vs baseline: 5.4916x; 1.3086x over previous
import functools

import jax
import jax.numpy as jnp
from jax import lax
from jax.experimental import pallas as pl
from jax.experimental.pallas import tpu as pltpu
from jax.experimental.pallas import tpu_sc as plsc

N = 10000
E = 320000
H = 128
EF = 16

NC = 2
NS = 16
NW = NC * NS

S = 5
ES = E // S
EPW = ES // NW
CSZ = 80
NCHUNK = EPW // CSZ
XP = 16

_f32 = jnp.float32
_i32 = jnp.int32


def _silu(z):
    return z / (1.0 + jnp.exp(-z))


def _prep_body(h_ref, w_ref, a_ref, b_ref):
    ab = jnp.dot(h_ref[...], w_ref[...], preferred_element_type=_f32)
    a_ref[...] = ab[:, :H]
    b_ref[...] = ab[:, H:]


def _prep(h, w_cat):
    return pl.pallas_call(
        _prep_body,
        out_shape=[jax.ShapeDtypeStruct((N, H), _f32),
                   jax.ShapeDtypeStruct((N, H), _f32)],
    )(h, w_cat)


def _make_sc_gather(slab):
    sbase = slab * ES

    def body(a_hbm, b_hbm, x_hbm, row_hbm, col_hbm, ga_hbm, gb_hbm, xc_hbm,
             rowv, colv, a0, b0, xc0, a1, b1, xc1, sem0, sem1):
        bufs = [(a0, b0, xc0), (a1, b1, xc1)]
        sems = [sem0, sem1]
        cid = lax.axis_index("c")
        sid = lax.axis_index("s")
        wid = cid * NS + sid

        pltpu.sync_copy(row_hbm.at[pl.ds(sbase + wid * EPW, EPW)], rowv)
        pltpu.sync_copy(col_hbm.at[pl.ds(sbase + wid * EPW, EPW)], colv)

        def start(c, k):
            bufa, bufb, bufxc = bufs[k]
            er = rowv.at[pl.ds(c * CSZ, CSZ)]
            ec = colv.at[pl.ds(c * CSZ, CSZ)]
            return [pltpu.async_copy(a_hbm.at[er], bufa, sems[k]),
                    pltpu.async_copy(b_hbm.at[ec], bufb, sems[k]),
                    pltpu.async_copy(x_hbm.at[ec], bufxc, sems[k])]

        def drain(c, k):
            bufa, bufb, bufxc = bufs[k]
            base = wid * EPW + c * CSZ
            pltpu.sync_copy(bufa, ga_hbm.at[pl.ds(base, CSZ)])
            pltpu.sync_copy(bufb, gb_hbm.at[pl.ds(base, CSZ)])
            pltpu.sync_copy(bufxc, xc_hbm.at[pl.ds(base, CSZ), pl.ds(0, XP)])

        def pair(k, carry):
            c0 = 2 * k
            cps0 = start(c0, 0)
            cps1 = start(c0 + 1, 1)
            for cp in cps0:
                cp.wait()
            drain(c0, 0)
            for cp in cps1:
                cp.wait()
            drain(c0 + 1, 1)
            return carry

        lax.fori_loop(0, NCHUNK // 2, pair, 0)
        cps = start(NCHUNK - 1, 0)
        for cp in cps:
            cp.wait()
        drain(NCHUNK - 1, 0)

    return pl.kernel(
        body,
        out_type=[jax.ShapeDtypeStruct((ES, H), _f32),
                  jax.ShapeDtypeStruct((ES, H), _f32),
                  jax.ShapeDtypeStruct((ES, H), _f32)],
        mesh=plsc.VectorSubcoreMesh(core_axis_name="c", subcore_axis_name="s",
                                    num_cores=NC, num_subcores=NS),
        compiler_params=pltpu.CompilerParams(use_tc_tiling_on_sc=False),
        scratch_types=[
            pltpu.VMEM((EPW,), _i32),
            pltpu.VMEM((EPW,), _i32),
            pltpu.VMEM((CSZ, H), _f32),
            pltpu.VMEM((CSZ, H), _f32),
            pltpu.VMEM((CSZ, XP), _f32),
            pltpu.VMEM((CSZ, H), _f32),
            pltpu.VMEM((CSZ, H), _f32),
            pltpu.VMEM((CSZ, XP), _f32),
            pltpu.SemaphoreType.DMA,
            pltpu.SemaphoreType.DMA,
        ],
    )


BE = 1280


def _mlp_body(ga, gb, ea, xc, wea, be1, we2, be2, wvab, bvab, wvb, bvb, out):
    z1 = ga[...] + gb[...] + be1[...] + jnp.dot(
        ea[...], wea[...], preferred_element_type=_f32)
    f1 = _silu(z1)
    z2 = jnp.dot(f1, we2[...], preferred_element_type=_f32) + be2[...]
    f2 = _silu(z2)
    t12 = _silu(jnp.dot(f2, wvab[...], preferred_element_type=_f32)
                + bvab[...])
    w12 = jnp.dot(t12, wvb[...], preferred_element_type=_f32) + bvb[...]
    w1 = w12[:, 0:1]
    w2 = w12[:, 1:2]
    d4 = jnp.concatenate(
        [xc[...][:, 0:3], jnp.ones((BE, 1), _f32)], axis=1)
    out[...] = jnp.concatenate(
        [d4 * w1, d4 * w2, jnp.zeros((BE, H - 8), _f32)], axis=1)


def _make_mlp(slab):
    nblk = ES // BE
    off = slab * nblk

    def full(shape):
        return pl.BlockSpec(shape, lambda i: (0,) * len(shape))

    return pl.pallas_call(
        _mlp_body,
        grid=(nblk,),
        in_specs=[
            pl.BlockSpec((BE, H), lambda i: (i, 0)),
            pl.BlockSpec((BE, H), lambda i: (i, 0)),
            pl.BlockSpec((BE, EF), lambda i: (i + off, 0)),
            pl.BlockSpec((BE, H), lambda i: (i, 0)),
            full((EF, H)), full((1, H)), full((H, H)), full((1, H)),
            full((H, 2 * H)), full((1, 2 * H)), full((2 * H, 2)),
            full((1, 2)),
        ],
        out_specs=pl.BlockSpec((BE, H), lambda i: (i, 0)),
        out_shape=jax.ShapeDtypeStruct((ES, H), _f32),
    )


def _scatter_body(row_hbm, vecs_hbm, z_hbm, out_hbm, idxv, vb0, vb1,
                  sem0, sem1, acc):
    cid = lax.axis_index("c")
    sid = lax.axis_index("s")
    wid = cid * NS + sid

    @pl.when(sid == 0)
    def _():
        pltpu.sync_copy(z_hbm, acc)

    plsc.subcore_barrier()

    for s in range(S):
        vec_hbm = vecs_hbm[s]
        pltpu.sync_copy(row_hbm.at[s, wid], idxv)

        def vload(c, buf, sem):
            base = wid * EPW + c * CSZ
            return pltpu.async_copy(
                vec_hbm.at[pl.ds(base, CSZ), pl.ds(0, 8)], buf, sem)

        def pair(k, carry):
            c0 = 2 * k
            cp0 = vload(c0, vb0, sem0)
            cp1 = vload(c0 + 1, vb1, sem1)
            cp0.wait()
            pltpu.sync_copy(vb0, acc.at[idxv.at[c0]], add=True)
            cp1.wait()
            pltpu.sync_copy(vb1, acc.at[idxv.at[c0 + 1]], add=True)
            return carry

        lax.fori_loop(0, NCHUNK // 2, pair, 0)
        cp = vload(NCHUNK - 1, vb0, sem0)
        cp.wait()
        pltpu.sync_copy(vb0, acc.at[idxv.at[NCHUNK - 1]], add=True)

    plsc.subcore_barrier()

    @pl.when(sid == 0)
    def _():
        pltpu.sync_copy(acc, out_hbm.at[cid])


@functools.partial(
    pl.kernel,
    out_type=jax.ShapeDtypeStruct((NC, N, 8), _f32),
    mesh=plsc.VectorSubcoreMesh(core_axis_name="c", subcore_axis_name="s",
                                num_cores=NC, num_subcores=NS),
    compiler_params=pltpu.CompilerParams(use_tc_tiling_on_sc=False),
    scratch_types=[
        pltpu.VMEM((NCHUNK, CSZ), _i32),
        pltpu.VMEM((CSZ, 8), _f32),
        pltpu.VMEM((CSZ, 8), _f32),
        pltpu.SemaphoreType.DMA,
        pltpu.SemaphoreType.DMA,
        pltpu.VMEM_SHARED((N, 8), _f32),
    ],
)
def _sc_scatter(row_hbm, v0, v1, v2, v3, v4, z_hbm, out_hbm,
                idxv, vb0, vb1, sem0, sem1, acc):
    _scatter_body(row_hbm, [v0, v1, v2, v3, v4], z_hbm, out_hbm,
                  idxv, vb0, vb1, sem0, sem1, acc)


BN = 400


def _final_body(p_ref, x_ref, out_ref):
    p = p_ref[...]
    xb = x_ref[...]
    v = p[0] + p[1]
    v1 = v[:, 0:3] - xb * v[:, 3:4]
    v2 = v[:, 4:7] - xb * v[:, 7:8]
    eps = jnp.float32(1e-12)
    n1 = jnp.sqrt(jnp.sum(v1 * v1, axis=1, keepdims=True))
    e1 = v1 / jnp.maximum(n1, eps)
    dot = jnp.sum(e1 * v2, axis=1, keepdims=True)
    pr = v2 - dot * e1
    n2 = jnp.sqrt(jnp.sum(pr * pr, axis=1, keepdims=True))
    e2 = pr / jnp.maximum(n2, eps)
    e1x, e1y, e1z = e1[:, 0:1], e1[:, 1:2], e1[:, 2:3]
    e2x, e2y, e2z = e2[:, 0:1], e2[:, 1:2], e2[:, 2:3]
    e3x = e1y * e2z - e1z * e2y
    e3y = e1z * e2x - e1x * e2z
    e3z = e1x * e2y - e1y * e2x
    out_ref[...] = jnp.concatenate(
        [e1x, e2x, e3x, e1y, e2y, e3y, e1z, e2z, e3z], axis=1)


def _final(partials, x):
    return pl.pallas_call(
        _final_body,
        grid=(N // BN,),
        in_specs=[pl.BlockSpec((NC, BN, 8), lambda i: (0, i, 0)),
                  pl.BlockSpec((BN, 3), lambda i: (i, 0))],
        out_specs=pl.BlockSpec((BN, 9), lambda i: (i, 0)),
        out_shape=jax.ShapeDtypeStruct((N, 9), _f32),
    )(partials, x)


def kernel(h, x, edge_index, edge_attr, W_e1, b_e1, W_e2, b_e2,
           W_v1a, b_v1a, W_v1b, b_v1b, W_v2a, b_v2a, W_v2b, b_v2b):
    row = edge_index[0].astype(_i32)
    col = edge_index[1].astype(_i32)
    x16 = jnp.pad(x, ((0, 0), (0, XP - 3)))

    w_cat = jnp.concatenate([W_e1[:H], W_e1[H:2 * H]], axis=1)
    a_tab, b_tab = _prep(h, w_cat)

    wea = W_e1[2 * H:]
    be1 = b_e1.reshape(1, H)
    be2 = b_e2.reshape(1, H)
    wvab = jnp.concatenate([W_v1a, W_v2a], axis=1)
    bvab = jnp.concatenate([b_v1a, b_v2a]).reshape(1, 2 * H)
    z1c = jnp.zeros((H, 1), _f32)
    wvb = jnp.concatenate(
        [jnp.concatenate([W_v1b, z1c], axis=1),
         jnp.concatenate([z1c, W_v2b], axis=1)], axis=0)
    bvb = jnp.stack([b_v1b[0], b_v2b[0]]).reshape(1, 2)

    vecs = []
    for s in range(S):
        ga, gb, xc = _make_sc_gather(s)(a_tab, b_tab, x16, row, col)
        vecs.append(_make_mlp(s)(ga, gb, edge_attr, xc,
                                 wea, be1, W_e2, be2, wvab, bvab, wvb, bvb))

    zeros8 = jnp.zeros((N, 8), _f32)
    partials = _sc_scatter(row.reshape(S, NW, NCHUNK, CSZ), *vecs, zeros8)

    out9 = _final(partials, x)
    return out9.reshape(N, 3, 3)


# speedup vs baseline: 5.6145x; 1.0224x over previous
import functools

import jax
import jax.numpy as jnp
from jax import lax
from jax.experimental import pallas as pl
from jax.experimental.pallas import tpu as pltpu
from jax.experimental.pallas import tpu_sc as plsc

N = 10000
E = 320000
H = 128
EF = 16

NC = 2
NS = 16
NW = NC * NS

S = 5
ES = E // S
EPW = ES // NW
CSZ = 80
NCHUNK = EPW // CSZ
XP = 16

_f32 = jnp.float32
_i32 = jnp.int32


def _silu(z):
    return z / (1.0 + jnp.exp(-z))


def _prep_body(h_ref, w_ref, a_ref, b_ref):
    ab = jnp.dot(h_ref[...], w_ref[...], preferred_element_type=_f32)
    a_ref[...] = ab[:, :H]
    b_ref[...] = ab[:, H:]


def _prep(h, w_cat):
    return pl.pallas_call(
        _prep_body,
        out_shape=[jax.ShapeDtypeStruct((N, H), _f32),
                   jax.ShapeDtypeStruct((N, H), _f32)],
    )(h, w_cat)


def _make_sc_gather(slab):
    sbase = slab * ES

    def body(a_hbm, b_hbm, x_hbm, row_hbm, col_hbm, ga_hbm, gb_hbm, xc_hbm,
             rowv, colv, a0, b0, xc0, a1, b1, xc1, sem0, sem1):
        bufs = [(a0, b0, xc0), (a1, b1, xc1)]
        sems = [sem0, sem1]
        cid = lax.axis_index("c")
        sid = lax.axis_index("s")
        wid = cid * NS + sid

        pltpu.sync_copy(row_hbm.at[pl.ds(sbase + wid * EPW, EPW)], rowv)
        pltpu.sync_copy(col_hbm.at[pl.ds(sbase + wid * EPW, EPW)], colv)

        def start(c, k):
            bufa, bufb, bufxc = bufs[k]
            er = rowv.at[pl.ds(c * CSZ, CSZ)]
            ec = colv.at[pl.ds(c * CSZ, CSZ)]
            return [pltpu.async_copy(a_hbm.at[er], bufa, sems[k]),
                    pltpu.async_copy(b_hbm.at[ec], bufb, sems[k]),
                    pltpu.async_copy(x_hbm.at[ec], bufxc, sems[k])]

        def drain(c, k):
            bufa, bufb, bufxc = bufs[k]
            base = wid * EPW + c * CSZ
            pltpu.sync_copy(bufa, ga_hbm.at[pl.ds(base, CSZ)])
            pltpu.sync_copy(bufb, gb_hbm.at[pl.ds(base, CSZ)])
            pltpu.sync_copy(bufxc, xc_hbm.at[pl.ds(base, CSZ), pl.ds(0, XP)])

        def pair(k, carry):
            c0 = 2 * k
            cps0 = start(c0, 0)
            cps1 = start(c0 + 1, 1)
            for cp in cps0:
                cp.wait()
            drain(c0, 0)
            for cp in cps1:
                cp.wait()
            drain(c0 + 1, 1)
            return carry

        lax.fori_loop(0, NCHUNK // 2, pair, 0)
        cps = start(NCHUNK - 1, 0)
        for cp in cps:
            cp.wait()
        drain(NCHUNK - 1, 0)

    return pl.kernel(
        body,
        out_type=[jax.ShapeDtypeStruct((ES, H), _f32),
                  jax.ShapeDtypeStruct((ES, H), _f32),
                  jax.ShapeDtypeStruct((ES, H), _f32)],
        mesh=plsc.VectorSubcoreMesh(core_axis_name="c", subcore_axis_name="s",
                                    num_cores=NC, num_subcores=NS),
        compiler_params=pltpu.CompilerParams(use_tc_tiling_on_sc=False),
        scratch_types=[
            pltpu.VMEM((EPW,), _i32),
            pltpu.VMEM((EPW,), _i32),
            pltpu.VMEM((CSZ, H), _f32),
            pltpu.VMEM((CSZ, H), _f32),
            pltpu.VMEM((CSZ, XP), _f32),
            pltpu.VMEM((CSZ, H), _f32),
            pltpu.VMEM((CSZ, H), _f32),
            pltpu.VMEM((CSZ, XP), _f32),
            pltpu.SemaphoreType.DMA,
            pltpu.SemaphoreType.DMA,
        ],
    )


BE = 1280


def _mlp_body(ga, gb, ea, xc, wea, be1, we2, be2, wvab, bvab, wvb, bvb, out):
    z1 = ga[...] + gb[...] + be1[...] + jnp.dot(
        ea[...], wea[...], preferred_element_type=_f32)
    f1 = _silu(z1)
    z2 = jnp.dot(f1, we2[...], preferred_element_type=_f32) + be2[...]
    f2 = _silu(z2)
    t12 = _silu(jnp.dot(f2, wvab[...], preferred_element_type=_f32)
                + bvab[...])
    w12 = jnp.dot(t12, wvb[...], preferred_element_type=_f32) + bvb[...]
    w1 = w12[:, 0:1]
    w2 = w12[:, 1:2]
    d4 = jnp.concatenate(
        [xc[...][:, 0:3], jnp.ones((BE, 1), _f32)], axis=1)
    out[...] = jnp.concatenate(
        [d4 * w1, d4 * w2, jnp.zeros((BE, H - 8), _f32)], axis=1)


def _make_mlp(slab):
    nblk = ES // BE
    off = slab * nblk

    def full(shape):
        return pl.BlockSpec(shape, lambda i: (0,) * len(shape))

    return pl.pallas_call(
        _mlp_body,
        grid=(nblk,),
        in_specs=[
            pl.BlockSpec((BE, H), lambda i: (i, 0)),
            pl.BlockSpec((BE, H), lambda i: (i, 0)),
            pl.BlockSpec((BE, EF), lambda i: (i + off, 0)),
            pl.BlockSpec((BE, H), lambda i: (i, 0)),
            full((EF, H)), full((1, H)), full((H, H)), full((1, H)),
            full((H, 2 * H)), full((1, 2 * H)), full((2 * H, 2)),
            full((1, 2)),
        ],
        out_specs=pl.BlockSpec((BE, H), lambda i: (i, 0)),
        out_shape=jax.ShapeDtypeStruct((ES, H), _f32),
    )


def _make_sc_scatter(slab_ids):
    nvec = len(slab_ids)

    def body(row_hbm, *args):
        vecs_hbm = args[:nvec]
        z_hbm, out_hbm, idxv, vb0, vb1, sem0, sem1, acc = args[nvec:]
        cid = lax.axis_index("c")
        sid = lax.axis_index("s")
        wid = cid * NS + sid

        @pl.when(sid == 0)
        def _():
            pltpu.sync_copy(z_hbm, acc)

        plsc.subcore_barrier()

        for vi, s in enumerate(slab_ids):
            vec_hbm = vecs_hbm[vi]
            pltpu.sync_copy(row_hbm.at[s, wid], idxv)

            def vload(c, buf, sem):
                base = wid * EPW + c * CSZ
                return pltpu.async_copy(
                    vec_hbm.at[pl.ds(base, CSZ), pl.ds(0, 8)], buf, sem)

            def pair(k, carry):
                c0 = 2 * k
                cp0 = vload(c0, vb0, sem0)
                cp1 = vload(c0 + 1, vb1, sem1)
                cp0.wait()
                pltpu.sync_copy(vb0, acc.at[idxv.at[c0]], add=True)
                cp1.wait()
                pltpu.sync_copy(vb1, acc.at[idxv.at[c0 + 1]], add=True)
                return carry

            lax.fori_loop(0, NCHUNK // 2, pair, 0)
            cp = vload(NCHUNK - 1, vb0, sem0)
            cp.wait()
            pltpu.sync_copy(vb0, acc.at[idxv.at[NCHUNK - 1]], add=True)

        plsc.subcore_barrier()

        @pl.when(sid == 0)
        def _():
            pltpu.sync_copy(acc, out_hbm.at[cid])

    return pl.kernel(
        body,
        out_type=jax.ShapeDtypeStruct((NC, N, 8), _f32),
        mesh=plsc.VectorSubcoreMesh(core_axis_name="c", subcore_axis_name="s",
                                    num_cores=NC, num_subcores=NS),
        compiler_params=pltpu.CompilerParams(use_tc_tiling_on_sc=False),
        scratch_types=[
            pltpu.VMEM((NCHUNK, CSZ), _i32),
            pltpu.VMEM((CSZ, 8), _f32),
            pltpu.VMEM((CSZ, 8), _f32),
            pltpu.SemaphoreType.DMA,
            pltpu.SemaphoreType.DMA,
            pltpu.VMEM_SHARED((N, 8), _f32),
        ],
    )


BN = 400


def _final_body(p_ref, q_ref, x_ref, out_ref):
    p = p_ref[...]
    q = q_ref[...]
    xb = x_ref[...]
    v = p[0] + p[1] + q[0] + q[1]
    v1 = v[:, 0:3] - xb * v[:, 3:4]
    v2 = v[:, 4:7] - xb * v[:, 7:8]
    eps = jnp.float32(1e-12)
    n1 = jnp.sqrt(jnp.sum(v1 * v1, axis=1, keepdims=True))
    e1 = v1 / jnp.maximum(n1, eps)
    dot = jnp.sum(e1 * v2, axis=1, keepdims=True)
    pr = v2 - dot * e1
    n2 = jnp.sqrt(jnp.sum(pr * pr, axis=1, keepdims=True))
    e2 = pr / jnp.maximum(n2, eps)
    e1x, e1y, e1z = e1[:, 0:1], e1[:, 1:2], e1[:, 2:3]
    e2x, e2y, e2z = e2[:, 0:1], e2[:, 1:2], e2[:, 2:3]
    e3x = e1y * e2z - e1z * e2y
    e3y = e1z * e2x - e1x * e2z
    e3z = e1x * e2y - e1y * e2x
    out_ref[...] = jnp.concatenate(
        [e1x, e2x, e3x, e1y, e2y, e3y, e1z, e2z, e3z], axis=1)


def _final(p0, p1, x):
    return pl.pallas_call(
        _final_body,
        grid=(N // BN,),
        in_specs=[pl.BlockSpec((NC, BN, 8), lambda i: (0, i, 0)),
                  pl.BlockSpec((NC, BN, 8), lambda i: (0, i, 0)),
                  pl.BlockSpec((BN, 3), lambda i: (i, 0))],
        out_specs=pl.BlockSpec((BN, 9), lambda i: (i, 0)),
        out_shape=jax.ShapeDtypeStruct((N, 9), _f32),
    )(p0, p1, x)


def kernel(h, x, edge_index, edge_attr, W_e1, b_e1, W_e2, b_e2,
           W_v1a, b_v1a, W_v1b, b_v1b, W_v2a, b_v2a, W_v2b, b_v2b):
    row = edge_index[0].astype(_i32)
    col = edge_index[1].astype(_i32)
    x16 = jnp.pad(x, ((0, 0), (0, XP - 3)))

    w_cat = jnp.concatenate([W_e1[:H], W_e1[H:2 * H]], axis=1)
    a_tab, b_tab = _prep(h, w_cat)

    wea = W_e1[2 * H:]
    be1 = b_e1.reshape(1, H)
    be2 = b_e2.reshape(1, H)
    wvab = jnp.concatenate([W_v1a, W_v2a], axis=1)
    bvab = jnp.concatenate([b_v1a, b_v2a]).reshape(1, 2 * H)
    z1c = jnp.zeros((H, 1), _f32)
    wvb = jnp.concatenate(
        [jnp.concatenate([W_v1b, z1c], axis=1),
         jnp.concatenate([z1c, W_v2b], axis=1)], axis=0)
    bvb = jnp.stack([b_v1b[0], b_v2b[0]]).reshape(1, 2)

    vecs = []
    for s in range(S):
        ga, gb, xc = _make_sc_gather(s)(a_tab, b_tab, x16, row, col)
        vecs.append(_make_mlp(s)(ga, gb, edge_attr, xc,
                                 wea, be1, W_e2, be2, wvab, bvab, wvb, bvb))

    zeros8 = jnp.zeros((N, 8), _f32)
    row4 = row.reshape(S, NW, NCHUNK, CSZ)
    p0 = _make_sc_scatter((0, 1, 2))(row4, vecs[0], vecs[1], vecs[2], zeros8)
    p1 = _make_sc_scatter((3, 4))(row4, vecs[3], vecs[4], zeros8)

    out9 = _final(p0, p1, x)
    return out9.reshape(N, 3, 3)


# speedup vs baseline: 6.2004x; 1.1043x over previous
import functools

import jax
import jax.numpy as jnp
from jax import lax
from jax.experimental import pallas as pl
from jax.experimental.pallas import tpu as pltpu
from jax.experimental.pallas import tpu_sc as plsc

N = 10000
E = 320000
H = 128
EF = 16

NC = 2
NS = 16
NW = NC * NS

S = 5
ES = E // S
EPW = ES // NW
CSZ = 80
NCHUNK = EPW // CSZ
XP = 16

_f32 = jnp.float32
_i32 = jnp.int32


def _silu(z):
    return z / (1.0 + jnp.exp(-z))


def _prep_body(h_ref, w_ref, a_ref, b_ref):
    ab = jnp.dot(h_ref[...], w_ref[...], preferred_element_type=_f32)
    a_ref[...] = ab[:, :H]
    b_ref[...] = ab[:, H:]


def _prep(h, w_cat):
    return pl.pallas_call(
        _prep_body,
        out_shape=[jax.ShapeDtypeStruct((N, H), _f32),
                   jax.ShapeDtypeStruct((N, H), _f32)],
    )(h, w_cat)


def _make_sc_gather(slab):
    sbase = slab * ES

    def body(a_hbm, b_hbm, x_hbm, row_hbm, col_hbm, ga_hbm, gb_hbm, xc_hbm,
             rowv, colv, a0, b0, xc0, a1, b1, xc1, sem0, sem1):
        bufs = [(a0, b0, xc0), (a1, b1, xc1)]
        sems = [sem0, sem1]
        cid = lax.axis_index("c")
        sid = lax.axis_index("s")
        wid = cid * NS + sid

        pltpu.sync_copy(row_hbm.at[pl.ds(sbase + wid * EPW, EPW)], rowv)
        pltpu.sync_copy(col_hbm.at[pl.ds(sbase + wid * EPW, EPW)], colv)

        def start(c, k):
            bufa, bufb, bufxc = bufs[k]
            er = rowv.at[pl.ds(c * CSZ, CSZ)]
            ec = colv.at[pl.ds(c * CSZ, CSZ)]
            return [pltpu.async_copy(a_hbm.at[er], bufa, sems[k]),
                    pltpu.async_copy(b_hbm.at[ec], bufb, sems[k]),
                    pltpu.async_copy(x_hbm.at[ec], bufxc, sems[k])]

        def drain(c, k):
            bufa, bufb, bufxc = bufs[k]
            base = wid * EPW + c * CSZ
            pltpu.sync_copy(bufa, ga_hbm.at[pl.ds(base, CSZ)])
            pltpu.sync_copy(bufb, gb_hbm.at[pl.ds(base, CSZ)])
            pltpu.sync_copy(bufxc, xc_hbm.at[pl.ds(base, CSZ), pl.ds(0, XP)])

        def pair(k, carry):
            c0 = 2 * k
            cps0 = start(c0, 0)
            cps1 = start(c0 + 1, 1)
            for cp in cps0:
                cp.wait()
            drain(c0, 0)
            for cp in cps1:
                cp.wait()
            drain(c0 + 1, 1)
            return carry

        lax.fori_loop(0, NCHUNK // 2, pair, 0)
        cps = start(NCHUNK - 1, 0)
        for cp in cps:
            cp.wait()
        drain(NCHUNK - 1, 0)

    return pl.kernel(
        body,
        out_type=[jax.ShapeDtypeStruct((ES, H), _f32),
                  jax.ShapeDtypeStruct((ES, H), _f32),
                  jax.ShapeDtypeStruct((ES, H), _f32)],
        mesh=plsc.VectorSubcoreMesh(core_axis_name="c", subcore_axis_name="s",
                                    num_cores=NC, num_subcores=NS),
        compiler_params=pltpu.CompilerParams(use_tc_tiling_on_sc=False),
        scratch_types=[
            pltpu.VMEM((EPW,), _i32),
            pltpu.VMEM((EPW,), _i32),
            pltpu.VMEM((CSZ, H), _f32),
            pltpu.VMEM((CSZ, H), _f32),
            pltpu.VMEM((CSZ, XP), _f32),
            pltpu.VMEM((CSZ, H), _f32),
            pltpu.VMEM((CSZ, H), _f32),
            pltpu.VMEM((CSZ, XP), _f32),
            pltpu.SemaphoreType.DMA,
            pltpu.SemaphoreType.DMA,
        ],
    )


BE = 3200


def _mlp_body(ga, gb, ea, xc, wea, be1, we2, be2, wvab, bvab, wvb, bvb, out):
    z1 = ga[...] + gb[...] + be1[...] + jnp.dot(
        ea[...], wea[...], preferred_element_type=_f32)
    f1 = _silu(z1)
    z2 = jnp.dot(f1, we2[...], preferred_element_type=_f32) + be2[...]
    f2 = _silu(z2)
    t12 = _silu(jnp.dot(f2, wvab[...], preferred_element_type=_f32)
                + bvab[...])
    w12 = jnp.dot(t12, wvb[...], preferred_element_type=_f32) + bvb[...]
    w1 = w12[:, 0:1]
    w2 = w12[:, 1:2]
    d4 = jnp.concatenate(
        [xc[...][:, 0:3], jnp.ones((BE, 1), _f32)], axis=1)
    out[...] = jnp.concatenate(
        [d4 * w1, d4 * w2, jnp.zeros((BE, H - 8), _f32)], axis=1)


def _make_mlp(slab):
    nblk = ES // BE
    off = slab * nblk

    def full(shape):
        return pl.BlockSpec(shape, lambda i: (0,) * len(shape))

    return pl.pallas_call(
        _mlp_body,
        grid=(nblk,),
        in_specs=[
            pl.BlockSpec((BE, H), lambda i: (i, 0)),
            pl.BlockSpec((BE, H), lambda i: (i, 0)),
            pl.BlockSpec((BE, EF), lambda i: (i + off, 0)),
            pl.BlockSpec((BE, H), lambda i: (i, 0)),
            full((EF, H)), full((1, H)), full((H, H)), full((1, H)),
            full((H, 2 * H)), full((1, 2 * H)), full((2 * H, 2)),
            full((1, 2)),
        ],
        out_specs=pl.BlockSpec((BE, H), lambda i: (i, 0)),
        out_shape=jax.ShapeDtypeStruct((ES, H), _f32),
    )


def _make_sc_scatter(slab_ids):
    nvec = len(slab_ids)

    def body(row_hbm, *args):
        vecs_hbm = args[:nvec]
        z_hbm, out_hbm, idxv, vb0, vb1, sem0, sem1, acc = args[nvec:]
        cid = lax.axis_index("c")
        sid = lax.axis_index("s")
        wid = cid * NS + sid

        @pl.when(sid == 0)
        def _():
            pltpu.sync_copy(z_hbm, acc)

        plsc.subcore_barrier()

        for vi, s in enumerate(slab_ids):
            vec_hbm = vecs_hbm[vi]
            pltpu.sync_copy(row_hbm.at[s, wid], idxv)

            def vload(c, buf, sem):
                base = wid * EPW + c * CSZ
                return pltpu.async_copy(
                    vec_hbm.at[pl.ds(base, CSZ), pl.ds(0, 8)], buf, sem)

            def pair(k, carry):
                c0 = 2 * k
                cp0 = vload(c0, vb0, sem0)
                cp1 = vload(c0 + 1, vb1, sem1)
                cp0.wait()
                pltpu.sync_copy(vb0, acc.at[idxv.at[c0]], add=True)
                cp1.wait()
                pltpu.sync_copy(vb1, acc.at[idxv.at[c0 + 1]], add=True)
                return carry

            lax.fori_loop(0, NCHUNK // 2, pair, 0)
            cp = vload(NCHUNK - 1, vb0, sem0)
            cp.wait()
            pltpu.sync_copy(vb0, acc.at[idxv.at[NCHUNK - 1]], add=True)

        plsc.subcore_barrier()

        @pl.when(sid == 0)
        def _():
            pltpu.sync_copy(acc, out_hbm.at[cid])

    return pl.kernel(
        body,
        out_type=jax.ShapeDtypeStruct((NC, N, 8), _f32),
        mesh=plsc.VectorSubcoreMesh(core_axis_name="c", subcore_axis_name="s",
                                    num_cores=NC, num_subcores=NS),
        compiler_params=pltpu.CompilerParams(use_tc_tiling_on_sc=False),
        scratch_types=[
            pltpu.VMEM((NCHUNK, CSZ), _i32),
            pltpu.VMEM((CSZ, 8), _f32),
            pltpu.VMEM((CSZ, 8), _f32),
            pltpu.SemaphoreType.DMA,
            pltpu.SemaphoreType.DMA,
            pltpu.VMEM_SHARED((N, 8), _f32),
        ],
    )


BN = 400


def _final_body(p_ref, q_ref, x_ref, out_ref):
    p = p_ref[...]
    q = q_ref[...]
    xb = x_ref[...]
    v = p[0] + p[1] + q[0] + q[1]
    v1 = v[:, 0:3] - xb * v[:, 3:4]
    v2 = v[:, 4:7] - xb * v[:, 7:8]
    eps = jnp.float32(1e-12)
    n1 = jnp.sqrt(jnp.sum(v1 * v1, axis=1, keepdims=True))
    e1 = v1 / jnp.maximum(n1, eps)
    dot = jnp.sum(e1 * v2, axis=1, keepdims=True)
    pr = v2 - dot * e1
    n2 = jnp.sqrt(jnp.sum(pr * pr, axis=1, keepdims=True))
    e2 = pr / jnp.maximum(n2, eps)
    e1x, e1y, e1z = e1[:, 0:1], e1[:, 1:2], e1[:, 2:3]
    e2x, e2y, e2z = e2[:, 0:1], e2[:, 1:2], e2[:, 2:3]
    e3x = e1y * e2z - e1z * e2y
    e3y = e1z * e2x - e1x * e2z
    e3z = e1x * e2y - e1y * e2x
    out_ref[...] = jnp.concatenate(
        [e1x, e2x, e3x, e1y, e2y, e3y, e1z, e2z, e3z], axis=1)


def _final(p0, p1, x):
    return pl.pallas_call(
        _final_body,
        grid=(N // BN,),
        in_specs=[pl.BlockSpec((NC, BN, 8), lambda i: (0, i, 0)),
                  pl.BlockSpec((NC, BN, 8), lambda i: (0, i, 0)),
                  pl.BlockSpec((BN, 3), lambda i: (i, 0))],
        out_specs=pl.BlockSpec((BN, 9), lambda i: (i, 0)),
        out_shape=jax.ShapeDtypeStruct((N, 9), _f32),
    )(p0, p1, x)


def kernel(h, x, edge_index, edge_attr, W_e1, b_e1, W_e2, b_e2,
           W_v1a, b_v1a, W_v1b, b_v1b, W_v2a, b_v2a, W_v2b, b_v2b):
    row = edge_index[0].astype(_i32)
    col = edge_index[1].astype(_i32)
    x16 = jnp.pad(x, ((0, 0), (0, XP - 3)))

    w_cat = jnp.concatenate([W_e1[:H], W_e1[H:2 * H]], axis=1)
    a_tab, b_tab = _prep(h, w_cat)

    wea = W_e1[2 * H:]
    be1 = b_e1.reshape(1, H)
    be2 = b_e2.reshape(1, H)
    wvab = jnp.concatenate([W_v1a, W_v2a], axis=1)
    bvab = jnp.concatenate([b_v1a, b_v2a]).reshape(1, 2 * H)
    z1c = jnp.zeros((H, 1), _f32)
    wvb = jnp.concatenate(
        [jnp.concatenate([W_v1b, z1c], axis=1),
         jnp.concatenate([z1c, W_v2b], axis=1)], axis=0)
    bvb = jnp.stack([b_v1b[0], b_v2b[0]]).reshape(1, 2)

    vecs = []
    for s in range(S):
        ga, gb, xc = _make_sc_gather(s)(a_tab, b_tab, x16, row, col)
        vecs.append(_make_mlp(s)(ga, gb, edge_attr, xc,
                                 wea, be1, W_e2, be2, wvab, bvab, wvb, bvb))

    zeros8 = jnp.zeros((N, 8), _f32)
    row4 = row.reshape(S, NW, NCHUNK, CSZ)
    p0 = _make_sc_scatter((0, 1, 2))(row4, vecs[0], vecs[1], vecs[2], zeros8)
    p1 = _make_sc_scatter((3, 4))(row4, vecs[3], vecs[4], zeros8)

    out9 = _final(p0, p1, x)
    return out9.reshape(N, 3, 3)
